# baseline (speedup 1.0000x reference)
import functools

import jax
import jax.numpy as jnp
import numpy as np
from jax import lax
from jax.experimental import pallas as pl
from jax.experimental.pallas import tpu as pltpu
from jax.experimental.pallas import tpu_sc as plsc

HIDDEN_SIZE = 2048
NUM_HEADS = 32
HEAD_DIM = HIDDEN_SIZE // NUM_HEADS
ROPE_THETA = 10000.0
MAX_POS = 4096
ATTENTION_SCALING = 1.0
BATCH = 2
SEQ = 4096

NC, NS = 2, 16
NW = NC * NS
W_PER_B = NW // BATCH
IDX_PER_W = SEQ // W_PER_B
CHUNK = 128
NCH = IDX_PER_W // CHUNK


def _build_tables():
    inv_freq = (1.0 / (ROPE_THETA ** (np.arange(0, HEAD_DIM, 2, dtype=np.float32) / HEAD_DIM))).astype(np.float32)
    t = np.arange(MAX_POS, dtype=np.float32)
    freqs = np.outer(t, inv_freq).astype(np.float32)
    emb = np.concatenate([freqs, freqs], axis=-1)
    cos = (np.cos(emb) * ATTENTION_SCALING).astype(np.float32)
    sin = (np.sin(emb) * ATTENTION_SCALING).astype(np.float32)
    return cos, sin


_COS_TABLE, _SIN_TABLE = _build_tables()


@functools.partial(
    pl.kernel,
    mesh=plsc.VectorSubcoreMesh(core_axis_name="c", subcore_axis_name="s"),
    out_type=(
        jax.ShapeDtypeStruct((BATCH, SEQ, HEAD_DIM), jnp.float32),
        jax.ShapeDtypeStruct((BATCH, SEQ, HEAD_DIM), jnp.float32),
    ),
    scratch_types=[
        pltpu.VMEM((NCH, CHUNK), jnp.int32),
        pltpu.VMEM((IDX_PER_W, HEAD_DIM), jnp.float32),
        pltpu.VMEM((IDX_PER_W, HEAD_DIM), jnp.float32),
        pltpu.SemaphoreType.DMA,
        pltpu.SemaphoreType.DMA,
        pltpu.SemaphoreType.DMA,
    ],
    compiler_params=pltpu.CompilerParams(
        use_tc_tiling_on_sc=False,
        disable_bounds_checks=True,
        disable_semaphore_checks=True,
    ),
)
def _rope_gather(cos_hbm, sin_hbm, pos_hbm, cos_out, sin_out,
                 idx_v, cos_rows, sin_rows, sem_a, sem_b, sem_st):
    wid = lax.axis_index("s") * NC + lax.axis_index("c")
    b = wid // W_PER_B
    row0 = (wid % W_PER_B) * IDX_PER_W
    for j in range(NCH):
        pltpu.sync_copy(pos_hbm.at[b, pl.ds(row0 + j * CHUNK, CHUNK)],
                        idx_v.at[j])
    gsems = (sem_a, sem_b)
    gathers = []
    for j in range(NCH):
        gathers.append((
            pltpu.async_copy(cos_hbm.at[idx_v.at[j]],
                             cos_rows.at[pl.ds(j * CHUNK, CHUNK)], gsems[j]),
            pltpu.async_copy(sin_hbm.at[idx_v.at[j]],
                             sin_rows.at[pl.ds(j * CHUNK, CHUNK)], gsems[j]),
        ))
    stores = []
    for j in range(NCH):
        gathers[j][0].wait()
        gathers[j][1].wait()
        stores.append(pltpu.async_copy(
            cos_rows.at[pl.ds(j * CHUNK, CHUNK)],
            cos_out.at[b, pl.ds(row0 + j * CHUNK, CHUNK)], sem_st))
        stores.append(pltpu.async_copy(
            sin_rows.at[pl.ds(j * CHUNK, CHUNK)],
            sin_out.at[b, pl.ds(row0 + j * CHUNK, CHUNK)], sem_st))
    for st in stores:
        st.wait()


def kernel(x, position_ids):
    cos_t = jnp.asarray(_COS_TABLE)
    sin_t = jnp.asarray(_SIN_TABLE)
    cos_o, sin_o = _rope_gather(cos_t, sin_t, position_ids)
    return cos_o.astype(x.dtype), sin_o.astype(x.dtype)
